# baseline (speedup 1.0000x reference)
import functools

import jax
import jax.numpy as jnp
from jax import lax
from jax.experimental import pallas as pl
from jax.experimental.pallas import tpu as pltpu
from jax.experimental.pallas import tpu_sc as plsc

B = 4096
VOCAB = 100000
D = 64
TOTAL = 204800

NC, NS, L = 2, 16, 16
NW = NC * NS
HALVES = 2
TPH = TOTAL // HALVES
TPW = TPH // NW
CH = 128
NCHUNK = TPW // CH
NCHP = 32
TW = 2 * D
KU = 16

_mesh = plsc.VectorSubcoreMesh(core_axis_name="c", subcore_axis_name="s",
                               num_cores=NC, num_subcores=NS)


def _make_sc_gather(half):
    hbase = half * TPH

    @functools.partial(
        pl.kernel,
        out_type=(
            jax.ShapeDtypeStruct((TPH, TW), jnp.float32),
            jax.ShapeDtypeStruct((NW, NCHP, CH), jnp.int32),
        ),
        mesh=_mesh,
        scratch_types=[
            pltpu.VMEM((TPW,), jnp.int32),
            pltpu.VMEM((TPW,), jnp.int32),
            pltpu.VMEM((3, CH, TW), jnp.float32),
            pltpu.VMEM((3, CH, TW), jnp.float32),
            pltpu.VMEM((3, KU, TW), jnp.float32),
            pltpu.VMEM((B + 1,), jnp.int32),
            pltpu.VMEM((B,), jnp.int32),
            pltpu.VMEM((NCHP, CH), jnp.int32),
            pltpu.SemaphoreType.DMA,
            pltpu.SemaphoreType.DMA,
            pltpu.SemaphoreType.DMA,
            pltpu.SemaphoreType.DMA,
            pltpu.SemaphoreType.DMA,
            pltpu.SemaphoreType.DMA,
            pltpu.SemaphoreType.DMA,
            pltpu.SemaphoreType.DMA,
            pltpu.SemaphoreType.DMA,
        ],
        compiler_params=pltpu.CompilerParams(needs_layout_passes=False),
        name=f"sc_gather_h{half}",
    )
    def _sc_gather(hist_hbm, cu_hbm, nodes_hbm, tab_hbm,
                   comb_hbm, seg_hbm,
                   hist_v, nidx_v, rows_v, rep_v, urows_v, cu_v, nodes_v,
                   segl_v,
                   sgv0, sgv1, sgv2, sgu0, sgu1, sgu2, swv0, swv1, swv2):
        cid = lax.axis_index("c")
        sid = lax.axis_index("s")
        wid = sid * NC + cid
        t0 = hbase + wid * TPW
        w0 = wid * TPW

        sgv = (sgv0, sgv1, sgv2)
        sgu = (sgu0, sgu1, sgu2)
        swv = (swv0, swv1, swv2)

        pltpu.sync_copy(cu_hbm, cu_v)
        pltpu.sync_copy(nodes_hbm, nodes_v)
        pltpu.sync_copy(hist_hbm.at[pl.ds(t0, TPW)], hist_v)

        lanes = jnp.arange(L, dtype=jnp.int32)
        zeros16 = jnp.zeros((L,), jnp.int32)

        @pl.loop(0, NCHP)
        def _zero(r):
            for c in range(CH // L):
                segl_v.at[r][pl.ds(c * L, L)] = zeros16

        def _bnd_body(it, acc):
            j = it * L + lanes + 1
            jvalid = j <= (B - 1)
            bnd = plsc.load_gather(cu_v, [jnp.where(jvalid, j, 0)])
            nxt = plsc.load_gather(cu_v, [jnp.where(jvalid, j + 1, 0)])
            last = bnd != nxt
            inr = jvalid & last & (bnd >= t0) & (bnd < t0 + TPW)
            pos = jnp.where(inr, bnd - t0, 0)
            plsc.store_scatter(segl_v, [pos // CH, pos % CH], j, mask=inr)
            below = jvalid & (bnd < t0)
            return acc + plsc.all_reduce_population_count(below)

        acc0 = jnp.zeros((L,), jnp.int32)
        sbase_vec = lax.fori_loop(0, B // L, _bnd_body, acc0)
        sbase = jnp.max(sbase_vec)

        def _cm_body(i, carry):
            r = i // (CH // L)
            c = i % (CH // L)
            v = segl_v.at[r][pl.ds(c * L, L)]
            cm = jnp.maximum(plsc.cummax(v), carry)
            segl_v.at[r][pl.ds(c * L, L)] = cm
            return jnp.max(cm)

        lax.fori_loop(0, TPW // L, _cm_body, sbase)
        pltpu.sync_copy(segl_v, seg_hbm.at[wid])

        @pl.loop(0, NCHP)
        def _nid(r):
            for c in range(CH // L):
                s16 = segl_v.at[r][pl.ds(c * L, L)]
                n16 = plsc.load_gather(nodes_v, [s16])
                nidx_v[pl.ds(r * CH + c * L, L)] = n16

        lanes_kf = jnp.arange(L, dtype=jnp.int32)

        def _span(g):
            s0 = jnp.min(segl_v.at[g][pl.ds(0, L)])
            slast = jnp.max(segl_v.at[g][pl.ds(CH - L, L)])
            return s0, slast - s0 + 1

        def _fast_idx(s0):
            ids = jnp.minimum(s0 + lanes_kf, B - 1)
            return plsc.load_gather(nodes_v, [ids])

        HC = CH // 2

        def _start_gathers(g, b):
            pltpu.async_copy(tab_hbm.at[hist_v.at[pl.ds(g * CH, HC)]],
                             rows_v.at[b].at[pl.ds(0, HC)], sgv[b])
            pltpu.async_copy(tab_hbm.at[hist_v.at[pl.ds(g * CH + HC, HC)]],
                             rows_v.at[b].at[pl.ds(HC, HC)], sgv[b])
            s0, span = _span(g)

            @pl.when(span <= KU)
            def _fast():
                pltpu.async_copy(tab_hbm.at[_fast_idx(s0)], urows_v.at[b],
                                 sgu[b])

            @pl.when(span > KU)
            def _slow():
                pltpu.async_copy(tab_hbm.at[nidx_v.at[pl.ds(g * CH, CH)]],
                                 rep_v.at[b], sgu[b])

        def _wait_gathers(g, b):
            pltpu.make_async_copy(tab_hbm.at[hist_v.at[pl.ds(g * CH, HC)]],
                                  rows_v.at[b].at[pl.ds(0, HC)],
                                  sgv[b]).wait()
            pltpu.make_async_copy(
                tab_hbm.at[hist_v.at[pl.ds(g * CH + HC, HC)]],
                rows_v.at[b].at[pl.ds(HC, HC)], sgv[b]).wait()
            s0, span = _span(g)

            @pl.when(span <= KU)
            def _fast():
                pltpu.make_async_copy(tab_hbm.at[_fast_idx(s0)],
                                      urows_v.at[b], sgu[b]).wait()

            @pl.when(span > KU)
            def _slow():
                pltpu.make_async_copy(
                    tab_hbm.at[nidx_v.at[pl.ds(g * CH, CH)]],
                    rep_v.at[b], sgu[b]).wait()

        def _merge(g, b):
            s0, span = _span(g)

            @pl.when(span <= KU)
            def _fast():
                @pl.loop(0, CH // L)
                def _grp(rr):
                    seg16 = segl_v.at[g][pl.ds(rr * L, L)]
                    k16 = seg16 - s0
                    r16 = rr * L + lanes_kf
                    for j in range(D):
                        col = jnp.full((L,), D + j, jnp.int32)
                        vals = plsc.load_gather(urows_v.at[b], [k16, col])
                        plsc.store_scatter(rows_v.at[b], [r16, col], vals)

            @pl.when(span > KU)
            def _slow():
                @pl.loop(0, CH)
                def _row(r):
                    for c in range(D // L):
                        off = D + c * L
                        rows_v.at[b].at[r][pl.ds(off, L)] = \
                            rep_v.at[b].at[r][pl.ds(off, L)]

        def _start_writes(g, b):
            pltpu.async_copy(rows_v.at[b],
                             comb_hbm.at[pl.ds(w0 + g * CH, CH)], swv[b])

        def _wait_writes(g, b):
            pltpu.make_async_copy(rows_v.at[b],
                                  comb_hbm.at[pl.ds(w0 + g * CH, CH)],
                                  swv[b]).wait()

        for b in range(2):
            _start_gathers(b, b)

        nsteps = (NCHUNK + 2) // 3

        @pl.loop(0, nsteps)
        def _chunk(gg):
            for k in range(3):
                g = gg * 3 + k

                @pl.when(g < NCHUNK)
                def _step():
                    _wait_gathers(g, k)
                    _merge(g, k)
                    _start_writes(g, k)
                    b2 = (k + 2) % 3

                    @pl.when(g >= 1)
                    def _drain_prev():
                        _wait_writes(g - 1, b2)

                    @pl.when(g + 2 < NCHUNK)
                    def _next():
                        _start_gathers(g + 2, b2)

        _wait_writes(NCHUNK - 1, (NCHUNK - 1) % 3)

    return _sc_gather


_sc_gather_h = tuple(_make_sc_gather(h) for h in range(HALVES))


_TB = 2048


def _mlp_body(x_ref, w1_ref, b1_ref, w2_ref, b2_ref,
              w3b_ref, wex_ref):
    x = x_ref[...]
    eint = x[:, :D]
    h = lax.dot_general(x, w1_ref[...], (((1,), (1,)), ((), ())),
                        preferred_element_type=jnp.float32) + b1_ref[...]
    h = jnp.maximum(h, 0.0)
    h = lax.dot_general(h, w2_ref[...], (((1,), (1,)), ((), ())),
                        preferred_element_type=jnp.float32) + b2_ref[...]
    h = jnp.maximum(h, 0.0)
    s = lax.dot_general(h, w3b_ref[...], (((1,), (1,)), ((), ())),
                        preferred_element_type=jnp.float32)
    ex = jnp.exp(s)
    wex_ref[...] = jnp.concatenate([eint * ex[:, :D], ex], axis=1)[:, :TW]


def _tc_mlp(x, w1, b1, w2, b2, w3b):
    grid = TPH // _TB
    return pl.pallas_call(
        _mlp_body,
        grid=(grid,),
        in_specs=[
            pl.BlockSpec((_TB, TW), lambda i: (i, 0)),
            pl.BlockSpec((D, 2 * D), lambda i: (0, 0)),
            pl.BlockSpec((1, D), lambda i: (0, 0)),
            pl.BlockSpec((D, D), lambda i: (0, 0)),
            pl.BlockSpec((1, D), lambda i: (0, 0)),
            pl.BlockSpec((D, D), lambda i: (0, 0)),
        ],
        out_specs=pl.BlockSpec((_TB, TW), lambda i: (i, 0)),
        out_shape=jax.ShapeDtypeStruct((TPH, TW), jnp.float32),
    )(x, w1, b1, w2, b2, w3b)


def _make_sc_scatter(half):
    @functools.partial(
        pl.kernel,
        out_type=jax.ShapeDtypeStruct((NC, B, TW), jnp.float32),
        mesh=_mesh,
        scratch_types=[
            pltpu.VMEM((NCHP, CH), jnp.int32),
            pltpu.VMEM((2, CH, TW), jnp.float32),
            pltpu.VMEM_SHARED((B, TW), jnp.float32),
            pltpu.SemaphoreType.DMA,
            pltpu.SemaphoreType.DMA,
        ],
        compiler_params=pltpu.CompilerParams(needs_layout_passes=False),
        name=f"sc_scatter_h{half}",
    )
    def _sc_scatter(seg_hbm, wex_hbm, zeros_hbm, out_hbm,
                    seg_v, wex_v, acc_sh, sl0, sl1):
        cid = lax.axis_index("c")
        sid = lax.axis_index("s")
        wid = sid * NC + cid
        stripe = B // NS
        sl = (sl0, sl1)

        pltpu.sync_copy(zeros_hbm.at[pl.ds(sid * stripe, stripe)],
                        acc_sh.at[pl.ds(sid * stripe, stripe)])
        plsc.subcore_barrier()

        pltpu.sync_copy(seg_hbm.at[wid], seg_v)

        def _load(g, b):
            base = wid * TPW + g * CH
            return pltpu.async_copy(wex_hbm.at[pl.ds(base, CH)],
                                    wex_v.at[b], sl[b])

        def _wait_load(g, b):
            base = wid * TPW + g * CH
            pltpu.make_async_copy(wex_hbm.at[pl.ds(base, CH)],
                                  wex_v.at[b], sl[b]).wait()

        _load(0, 0)

        @pl.loop(0, (NCHUNK + 1) // 2)
        def _chunk(gg):
            for b in range(2):
                g = gg * 2 + b

                @pl.when(g < NCHUNK)
                def _step():
                    @pl.when(g + 1 < NCHUNK)
                    def _next():
                        _load(g + 1, 1 - b)

                    _wait_load(g, b)
                    pltpu.sync_copy(wex_v.at[b], acc_sh.at[seg_v.at[g]],
                                    add=True)

        plsc.subcore_barrier()
        pltpu.sync_copy(acc_sh.at[pl.ds(sid * stripe, stripe)],
                        out_hbm.at[cid].at[pl.ds(sid * stripe, stripe)])

    return _sc_scatter


_sc_scatter_h = tuple(_make_sc_scatter(h) for h in range(HALVES))


_DB = 512


def _div_body(p0_ref, p1_ref, p2_ref, p3_ref, o_ref):
    s = p0_ref[...] + p1_ref[...] + p2_ref[...] + p3_ref[...]
    den = s[:, D:D + 1]
    o_ref[...] = jnp.where(den > 0.0, s[:, :D] / den, 0.0)


def _tc_combine(parts):
    return pl.pallas_call(
        _div_body,
        grid=(B // _DB,),
        in_specs=[pl.BlockSpec((_DB, TW), lambda i: (i, 0))
                  for _ in range(4)],
        out_specs=pl.BlockSpec((_DB, D), lambda i: (i, 0)),
        out_shape=jax.ShapeDtypeStruct((B, D), jnp.float32),
    )(*parts)


def kernel(nodes, history_flat, cu_seqlens, v_to_e_weight, u_to_e_weight,
           att1_w, att1_b, att2_w, att2_b, att3_w, att3_b):
    hist = history_flat.astype(jnp.int32)
    cu = cu_seqlens.astype(jnp.int32)
    nds = nodes.astype(jnp.int32)

    tab = jnp.concatenate([v_to_e_weight, u_to_e_weight], axis=1)
    w3b = jnp.broadcast_to(att3_w, (D, D))
    b1 = att1_b.reshape(1, D)
    b2 = att2_b.reshape(1, D)
    zeros = jnp.zeros((B, TW), jnp.float32)

    parts = []
    for h in range(HALVES):
        comb, seg = _sc_gather_h[h](hist, cu, nds, tab)
        wex = _tc_mlp(comb, att1_w, b1, att2_w, b2, w3b)
        p = _sc_scatter_h[h](seg, wex, zeros)
        parts.extend([p[0], p[1]])
    return _tc_combine(parts)


# speedup vs baseline: 1.0023x; 1.0023x over previous
import functools

import jax
import jax.numpy as jnp
from jax import lax
from jax.experimental import pallas as pl
from jax.experimental.pallas import tpu as pltpu
from jax.experimental.pallas import tpu_sc as plsc

B = 4096
VOCAB = 100000
D = 64
TOTAL = 204800

NC, NS, L = 2, 16, 16
NW = NC * NS
HALVES = 2
TPH = TOTAL // HALVES
TPW = TPH // NW
CH = 128
NCHUNK = TPW // CH
NCHP = 32
TW = 2 * D
KU = 16

_mesh = plsc.VectorSubcoreMesh(core_axis_name="c", subcore_axis_name="s",
                               num_cores=NC, num_subcores=NS)


def _make_sc_gather(half):
    hbase = half * TPH

    @functools.partial(
        pl.kernel,
        out_type=(
            jax.ShapeDtypeStruct((TPH, TW), jnp.float32),
            jax.ShapeDtypeStruct((NW, NCHP, CH), jnp.int32),
        ),
        mesh=_mesh,
        scratch_types=[
            pltpu.VMEM((TPW,), jnp.int32),
            pltpu.VMEM((TPW,), jnp.int32),
            pltpu.VMEM((3, CH, TW), jnp.float32),
            pltpu.VMEM((3, CH, TW), jnp.float32),
            pltpu.VMEM((3, KU, TW), jnp.float32),
            pltpu.VMEM((B + 1,), jnp.int32),
            pltpu.VMEM((B,), jnp.int32),
            pltpu.VMEM((NCHP, CH), jnp.int32),
            pltpu.SemaphoreType.DMA,
            pltpu.SemaphoreType.DMA,
            pltpu.SemaphoreType.DMA,
            pltpu.SemaphoreType.DMA,
            pltpu.SemaphoreType.DMA,
            pltpu.SemaphoreType.DMA,
            pltpu.SemaphoreType.DMA,
            pltpu.SemaphoreType.DMA,
            pltpu.SemaphoreType.DMA,
        ],
        compiler_params=pltpu.CompilerParams(needs_layout_passes=False),
        name=f"sc_gather_h{half}",
    )
    def _sc_gather(hist_hbm, cu_hbm, nodes_hbm, tab_hbm,
                   comb_hbm, seg_hbm,
                   hist_v, nidx_v, rows_v, rep_v, urows_v, cu_v, nodes_v,
                   segl_v,
                   sgv0, sgv1, sgv2, sgu0, sgu1, sgu2, swv0, swv1, swv2):
        cid = lax.axis_index("c")
        sid = lax.axis_index("s")
        wid = sid * NC + cid
        t0 = hbase + wid * TPW
        w0 = wid * TPW

        sgv = (sgv0, sgv1, sgv2)
        sgu = (sgu0, sgu1, sgu2)
        swv = (swv0, swv1, swv2)

        pltpu.sync_copy(cu_hbm, cu_v)
        pltpu.sync_copy(nodes_hbm, nodes_v)
        pltpu.sync_copy(hist_hbm.at[pl.ds(t0, TPW)], hist_v)

        lanes = jnp.arange(L, dtype=jnp.int32)
        zeros16 = jnp.zeros((L,), jnp.int32)

        @pl.loop(0, NCHP)
        def _zero(r):
            for c in range(CH // L):
                segl_v.at[r][pl.ds(c * L, L)] = zeros16

        def _bnd_body(it, acc):
            j = it * L + lanes + 1
            jvalid = j <= (B - 1)
            bnd = plsc.load_gather(cu_v, [jnp.where(jvalid, j, 0)])
            nxt = plsc.load_gather(cu_v, [jnp.where(jvalid, j + 1, 0)])
            last = bnd != nxt
            inr = jvalid & last & (bnd >= t0) & (bnd < t0 + TPW)
            pos = jnp.where(inr, bnd - t0, 0)
            plsc.store_scatter(segl_v, [pos // CH, pos % CH], j, mask=inr)
            below = jvalid & (bnd < t0)
            return acc + plsc.all_reduce_population_count(below)

        acc0 = jnp.zeros((L,), jnp.int32)
        sbase_vec = lax.fori_loop(0, B // L, _bnd_body, acc0)
        sbase = jnp.max(sbase_vec)

        def _cm_body(i, carry):
            r = i // (CH // L)
            c = i % (CH // L)
            v = segl_v.at[r][pl.ds(c * L, L)]
            cm = jnp.maximum(plsc.cummax(v), carry)
            segl_v.at[r][pl.ds(c * L, L)] = cm
            return jnp.max(cm)

        lax.fori_loop(0, TPW // L, _cm_body, sbase)
        pltpu.sync_copy(segl_v, seg_hbm.at[wid])

        @pl.loop(0, NCHP)
        def _nid(r):
            for c in range(CH // L):
                s16 = segl_v.at[r][pl.ds(c * L, L)]
                n16 = plsc.load_gather(nodes_v, [s16])
                nidx_v[pl.ds(r * CH + c * L, L)] = n16

        lanes_kf = jnp.arange(L, dtype=jnp.int32)

        def _span(g):
            s0 = jnp.min(segl_v.at[g][pl.ds(0, L)])
            slast = jnp.max(segl_v.at[g][pl.ds(CH - L, L)])
            return s0, slast - s0 + 1

        def _fast_idx(s0):
            ids = jnp.minimum(s0 + lanes_kf, B - 1)
            return plsc.load_gather(nodes_v, [ids])

        def _start_gathers(g, b):
            pltpu.async_copy(tab_hbm.at[hist_v.at[pl.ds(g * CH, CH)]],
                             rows_v.at[b], sgv[b])
            s0, span = _span(g)

            @pl.when(span <= KU)
            def _fast():
                pltpu.async_copy(tab_hbm.at[_fast_idx(s0)], urows_v.at[b],
                                 sgu[b])

            @pl.when(span > KU)
            def _slow():
                pltpu.async_copy(tab_hbm.at[nidx_v.at[pl.ds(g * CH, CH)]],
                                 rep_v.at[b], sgu[b])

        def _wait_gathers(g, b):
            pltpu.make_async_copy(tab_hbm.at[hist_v.at[pl.ds(g * CH, CH)]],
                                  rows_v.at[b], sgv[b]).wait()
            s0, span = _span(g)

            @pl.when(span <= KU)
            def _fast():
                pltpu.make_async_copy(tab_hbm.at[_fast_idx(s0)],
                                      urows_v.at[b], sgu[b]).wait()

            @pl.when(span > KU)
            def _slow():
                pltpu.make_async_copy(
                    tab_hbm.at[nidx_v.at[pl.ds(g * CH, CH)]],
                    rep_v.at[b], sgu[b]).wait()

        def _merge(g, b):
            s0, span = _span(g)

            @pl.when(span <= KU)
            def _fast():
                @pl.loop(0, CH // L)
                def _grp(rr):
                    seg16 = segl_v.at[g][pl.ds(rr * L, L)]
                    k16 = seg16 - s0
                    r16 = rr * L + lanes_kf
                    for j in range(D):
                        col = jnp.full((L,), D + j, jnp.int32)
                        vals = plsc.load_gather(urows_v.at[b], [k16, col])
                        plsc.store_scatter(rows_v.at[b], [r16, col], vals)

            @pl.when(span > KU)
            def _slow():
                @pl.loop(0, CH)
                def _row(r):
                    for c in range(D // L):
                        off = D + c * L
                        rows_v.at[b].at[r][pl.ds(off, L)] = \
                            rep_v.at[b].at[r][pl.ds(off, L)]

        def _start_writes(g, b):
            pltpu.async_copy(rows_v.at[b],
                             comb_hbm.at[pl.ds(w0 + g * CH, CH)], swv[b])

        def _wait_writes(g, b):
            pltpu.make_async_copy(rows_v.at[b],
                                  comb_hbm.at[pl.ds(w0 + g * CH, CH)],
                                  swv[b]).wait()

        for b in range(2):
            _start_gathers(b, b)

        nsteps = (NCHUNK + 2) // 3

        @pl.loop(0, nsteps)
        def _chunk(gg):
            for k in range(3):
                g = gg * 3 + k

                @pl.when(g < NCHUNK)
                def _step():
                    _wait_gathers(g, k)
                    _merge(g, k)
                    _start_writes(g, k)
                    b2 = (k + 2) % 3

                    @pl.when(g >= 1)
                    def _drain_prev():
                        _wait_writes(g - 1, b2)

                    @pl.when(g + 2 < NCHUNK)
                    def _next():
                        _start_gathers(g + 2, b2)

        _wait_writes(NCHUNK - 1, (NCHUNK - 1) % 3)

    return _sc_gather


_sc_gather_h = tuple(_make_sc_gather(h) for h in range(HALVES))


_TB = 2048


def _mlp_body(x_ref, w1_ref, b1_ref, w2_ref, b2_ref,
              w3b_ref, wex_ref):
    x = x_ref[...]
    eint = x[:, :D]
    h = lax.dot_general(x, w1_ref[...], (((1,), (1,)), ((), ())),
                        preferred_element_type=jnp.float32) + b1_ref[...]
    h = jnp.maximum(h, 0.0)
    h = lax.dot_general(h, w2_ref[...], (((1,), (1,)), ((), ())),
                        preferred_element_type=jnp.float32) + b2_ref[...]
    h = jnp.maximum(h, 0.0)
    s = lax.dot_general(h, w3b_ref[...], (((1,), (1,)), ((), ())),
                        preferred_element_type=jnp.float32)
    ex = jnp.exp(s)
    wex_ref[...] = jnp.concatenate([eint * ex[:, :D], ex], axis=1)[:, :TW]


def _tc_mlp(x, w1, b1, w2, b2, w3b):
    grid = TPH // _TB
    return pl.pallas_call(
        _mlp_body,
        grid=(grid,),
        in_specs=[
            pl.BlockSpec((_TB, TW), lambda i: (i, 0)),
            pl.BlockSpec((D, 2 * D), lambda i: (0, 0)),
            pl.BlockSpec((1, D), lambda i: (0, 0)),
            pl.BlockSpec((D, D), lambda i: (0, 0)),
            pl.BlockSpec((1, D), lambda i: (0, 0)),
            pl.BlockSpec((D, D), lambda i: (0, 0)),
        ],
        out_specs=pl.BlockSpec((_TB, TW), lambda i: (i, 0)),
        out_shape=jax.ShapeDtypeStruct((TPH, TW), jnp.float32),
    )(x, w1, b1, w2, b2, w3b)


def _make_sc_scatter(half):
    @functools.partial(
        pl.kernel,
        out_type=jax.ShapeDtypeStruct((NC, B, TW), jnp.float32),
        mesh=_mesh,
        scratch_types=[
            pltpu.VMEM((NCHP, CH), jnp.int32),
            pltpu.VMEM((2, CH, TW), jnp.float32),
            pltpu.VMEM_SHARED((B, TW), jnp.float32),
            pltpu.SemaphoreType.DMA,
            pltpu.SemaphoreType.DMA,
        ],
        compiler_params=pltpu.CompilerParams(needs_layout_passes=False),
        name=f"sc_scatter_h{half}",
    )
    def _sc_scatter(seg_hbm, wex_hbm, zeros_hbm, out_hbm,
                    seg_v, wex_v, acc_sh, sl0, sl1):
        cid = lax.axis_index("c")
        sid = lax.axis_index("s")
        wid = sid * NC + cid
        stripe = B // NS
        sl = (sl0, sl1)

        pltpu.sync_copy(zeros_hbm.at[pl.ds(sid * stripe, stripe)],
                        acc_sh.at[pl.ds(sid * stripe, stripe)])
        plsc.subcore_barrier()

        pltpu.sync_copy(seg_hbm.at[wid], seg_v)

        def _load(g, b):
            base = wid * TPW + g * CH
            return pltpu.async_copy(wex_hbm.at[pl.ds(base, CH)],
                                    wex_v.at[b], sl[b])

        def _wait_load(g, b):
            base = wid * TPW + g * CH
            pltpu.make_async_copy(wex_hbm.at[pl.ds(base, CH)],
                                  wex_v.at[b], sl[b]).wait()

        _load(0, 0)

        @pl.loop(0, (NCHUNK + 1) // 2)
        def _chunk(gg):
            for b in range(2):
                g = gg * 2 + b

                @pl.when(g < NCHUNK)
                def _step():
                    @pl.when(g + 1 < NCHUNK)
                    def _next():
                        _load(g + 1, 1 - b)

                    _wait_load(g, b)
                    pltpu.sync_copy(wex_v.at[b], acc_sh.at[seg_v.at[g]],
                                    add=True)

        plsc.subcore_barrier()
        pltpu.sync_copy(acc_sh.at[pl.ds(sid * stripe, stripe)],
                        out_hbm.at[cid].at[pl.ds(sid * stripe, stripe)])

    return _sc_scatter


_sc_scatter_h = tuple(_make_sc_scatter(h) for h in range(HALVES))


_DB = 512


def _div_body(p0_ref, p1_ref, p2_ref, p3_ref, o_ref):
    s = p0_ref[...] + p1_ref[...] + p2_ref[...] + p3_ref[...]
    den = s[:, D:D + 1]
    o_ref[...] = jnp.where(den > 0.0, s[:, :D] / den, 0.0)


def _tc_combine(parts):
    return pl.pallas_call(
        _div_body,
        grid=(B // _DB,),
        in_specs=[pl.BlockSpec((_DB, TW), lambda i: (i, 0))
                  for _ in range(4)],
        out_specs=pl.BlockSpec((_DB, D), lambda i: (i, 0)),
        out_shape=jax.ShapeDtypeStruct((B, D), jnp.float32),
    )(*parts)


def kernel(nodes, history_flat, cu_seqlens, v_to_e_weight, u_to_e_weight,
           att1_w, att1_b, att2_w, att2_b, att3_w, att3_b):
    hist = history_flat.astype(jnp.int32)
    cu = cu_seqlens.astype(jnp.int32)
    nds = nodes.astype(jnp.int32)

    tab = jnp.concatenate([v_to_e_weight, u_to_e_weight], axis=1)
    w3b = jnp.broadcast_to(att3_w, (D, D))
    b1 = att1_b.reshape(1, D)
    b2 = att2_b.reshape(1, D)
    zeros = jnp.zeros((B, TW), jnp.float32)

    parts = []
    for h in range(HALVES):
        comb, seg = _sc_gather_h[h](hist, cu, nds, tab)
        wex = _tc_mlp(comb, att1_w, b1, att2_w, b2, w3b)
        p = _sc_scatter_h[h](seg, wex, zeros)
        parts.extend([p[0], p[1]])
    return _tc_combine(parts)
